# baseline (speedup 1.0000x reference)
import functools

import jax
import jax.numpy as jnp
from jax import lax
from jax.experimental import pallas as pl
from jax.experimental.pallas import tpu as pltpu
from jax.experimental.pallas import tpu_sc as plsc

TIME = 288
FEATURES = 64
B, T, N, C = 64, 12, 4096, 3

NUM_CORES = 2
NUM_SUBCORES = 16
NUM_WORKERS = NUM_CORES * NUM_SUBCORES
B_PER_W = B // NUM_WORKERS
LANES = 16
NCHUNKS = N // LANES
FBLK = 8
NBLKS = FEATURES // FBLK


def _body(xr_hbm, td_hbm, tw_hbm, out_hbm,
          td_v, tw_v, xb_v, idxd_v, idxw_v, row_v, sem0, sem1):
    sems = (sem0, sem1)
    wid = lax.axis_index("s") * NUM_CORES + lax.axis_index("c")
    iota3 = lax.iota(jnp.int32, LANES) * C

    pltpu.sync_copy(td_hbm, td_v)
    pltpu.sync_copy(tw_hbm, tw_v)

    for b_local in range(B_PER_W):
        b = wid * B_PER_W + b_local

        pltpu.sync_copy(xr_hbm.at[b, T - 1], xb_v)

        @plsc.parallel_loop(0, NCHUNKS, unroll=4)
        def idx_body(i):
            nd = i * (LANES * C) + iota3
            dayv = plsc.load_gather(xb_v, [nd + 1])
            weekv = plsc.load_gather(xb_v, [nd + 2])
            d = jnp.clip((dayv * float(TIME)).astype(jnp.int32), 0, TIME - 1)
            w = jnp.clip(weekv.astype(jnp.int32), 0, 6)
            sl = pl.ds(i * LANES, LANES)
            idxd_v[sl] = d
            idxw_v[sl] = w

        pending = {0: [], 1: []}
        for fblk in range(NBLKS):
            ph = fblk % 2
            for dsc in pending[ph]:
                dsc.wait()
            pending[ph] = []

            @plsc.parallel_loop(0, NCHUNKS, unroll=2)
            def gather_body(i, ph=ph, fblk=fblk):
                sl = pl.ds(i * LANES, LANES)
                dvec = idxd_v[sl]
                wvec = idxw_v[sl]
                for j in range(FBLK):
                    f = fblk * FBLK + j
                    dayv = plsc.load_gather(td_v, [dvec + f * TIME])
                    weekv = plsc.load_gather(tw_v, [wvec + f * 7])
                    row_v[ph, j, sl] = dayv + weekv

            for j in range(FBLK):
                dsc = pltpu.async_copy(
                    row_v.at[ph, j], out_hbm.at[b, fblk * FBLK + j], sems[ph])
                pending[ph].append(dsc)

        for ph in (0, 1):
            for dsc in pending[ph]:
                dsc.wait()


def kernel(x, time_day, time_week):
    xr = x.reshape(B, T, N * C)
    td = time_day.T.reshape(-1)
    tw = time_week.T.reshape(-1)

    mesh = plsc.VectorSubcoreMesh(
        core_axis_name="c", subcore_axis_name="s",
        num_cores=NUM_CORES, num_subcores=NUM_SUBCORES)
    run = functools.partial(
        pl.kernel,
        out_type=jax.ShapeDtypeStruct((B, FEATURES, N), jnp.float32),
        mesh=mesh,
        compiler_params=pltpu.CompilerParams(needs_layout_passes=False),
        scratch_types=[
            pltpu.VMEM((FEATURES * TIME,), jnp.float32),
            pltpu.VMEM((FEATURES * 7,), jnp.float32),
            pltpu.VMEM((N * C,), jnp.float32),
            pltpu.VMEM((N,), jnp.int32),
            pltpu.VMEM((N,), jnp.int32),
            pltpu.VMEM((2, FBLK, N), jnp.float32),
            pltpu.SemaphoreType.DMA,
            pltpu.SemaphoreType.DMA,
        ],
    )(_body)
    out = run(xr, td, tw)
    return out[..., None]


# speedup vs baseline: 2.1856x; 2.1856x over previous
import functools

import jax
import jax.numpy as jnp
from jax import lax
from jax.experimental import pallas as pl
from jax.experimental.pallas import tpu as pltpu
from jax.experimental.pallas import tpu_sc as plsc

TIME = 288
FEATURES = 64
B, T, N, C = 64, 12, 4096, 3

NUM_CORES = 2
NUM_SUBCORES = 16
NUM_WORKERS = NUM_CORES * NUM_SUBCORES
B_PER_W = B // NUM_WORKERS
LANES = 16
NCHUNKS = N // LANES
FBLK = 8
NBLKS = FEATURES // FBLK


def _body(xs_hbm, td_hbm, tw_hbm, out_hbm,
          td_v, tw_v, xd_v, xw_v, idxd_v, idxw_v, row_v, sem0, sem1):
    sems = (sem0, sem1)
    wid = lax.axis_index("s") * NUM_CORES + lax.axis_index("c")

    pltpu.sync_copy(td_hbm, td_v)
    pltpu.sync_copy(tw_hbm, tw_v)

    for b_local in range(B_PER_W):
        b = wid * B_PER_W + b_local

        pltpu.sync_copy(xs_hbm.at[b, 0], xd_v)
        pltpu.sync_copy(xs_hbm.at[b, 1], xw_v)

        @plsc.parallel_loop(0, NCHUNKS, unroll=4)
        def idx_body(i):
            sl = pl.ds(i * LANES, LANES)
            dayv = xd_v[sl]
            weekv = xw_v[sl]
            d = jnp.clip((dayv * float(TIME)).astype(jnp.int32), 0, TIME - 1)
            w = jnp.clip(weekv.astype(jnp.int32), 0, 6)
            idxd_v[sl] = d
            idxw_v[sl] = w

        pending = {0: [], 1: []}
        for fblk in range(NBLKS):
            ph = fblk % 2
            for dsc in pending[ph]:
                dsc.wait()
            pending[ph] = []

            @plsc.parallel_loop(0, NCHUNKS, unroll=2)
            def gather_body(i, ph=ph, fblk=fblk):
                sl = pl.ds(i * LANES, LANES)
                dvec = idxd_v[sl]
                wvec = idxw_v[sl]
                for j in range(FBLK):
                    f = fblk * FBLK + j
                    dayv = plsc.load_gather(td_v, [dvec + f * TIME])
                    weekv = plsc.load_gather(tw_v, [wvec + f * 7])
                    row_v[ph, j, sl] = dayv + weekv

            for j in range(FBLK):
                dsc = pltpu.async_copy(
                    row_v.at[ph, j], out_hbm.at[b, fblk * FBLK + j], sems[ph])
                pending[ph].append(dsc)

        for ph in (0, 1):
            for dsc in pending[ph]:
                dsc.wait()


def kernel(x, time_day, time_week):
    xs = jnp.transpose(x[:, -1, :, 1:3], (0, 2, 1))
    td = time_day.T.reshape(-1)
    tw = time_week.T.reshape(-1)

    mesh = plsc.VectorSubcoreMesh(
        core_axis_name="c", subcore_axis_name="s",
        num_cores=NUM_CORES, num_subcores=NUM_SUBCORES)
    run = functools.partial(
        pl.kernel,
        out_type=jax.ShapeDtypeStruct((B, FEATURES, N), jnp.float32),
        mesh=mesh,
        compiler_params=pltpu.CompilerParams(needs_layout_passes=False),
        scratch_types=[
            pltpu.VMEM((FEATURES * TIME,), jnp.float32),
            pltpu.VMEM((FEATURES * 7,), jnp.float32),
            pltpu.VMEM((N,), jnp.float32),
            pltpu.VMEM((N,), jnp.float32),
            pltpu.VMEM((N,), jnp.int32),
            pltpu.VMEM((N,), jnp.int32),
            pltpu.VMEM((2, FBLK, N), jnp.float32),
            pltpu.SemaphoreType.DMA,
            pltpu.SemaphoreType.DMA,
        ],
    )(_body)
    out = run(xs, td, tw)
    return out[..., None]


# speedup vs baseline: 3.6139x; 1.6535x over previous
import functools

import jax
import jax.numpy as jnp
from jax import lax
from jax.experimental import pallas as pl
from jax.experimental.pallas import tpu as pltpu
from jax.experimental.pallas import tpu_sc as plsc

TIME = 288
FEATURES = 64
B, T, N, C = 64, 12, 4096, 3

NUM_CORES = 2
NUM_SUBCORES = 16
NUM_WORKERS = NUM_CORES * NUM_SUBCORES
B_PER_W = B // NUM_WORKERS
LANES = 16
NCHUNKS = N // LANES
FBLK = 8
NBLKS = FEATURES // FBLK


def _body(xs_hbm, td_hbm, tw_hbm, out_hbm,
          td_v, tw_v, xd_v, xw_v, idxd_v, idxw_v, row_v, sem0, sem1):
    sems = (sem0, sem1)
    wid = lax.axis_index("s") * NUM_CORES + lax.axis_index("c")

    pltpu.sync_copy(td_hbm, td_v)
    pltpu.sync_copy(tw_hbm, tw_v)

    for b_local in range(B_PER_W):
        b = wid * B_PER_W + b_local

        pltpu.sync_copy(xs_hbm.at[b, 0], xd_v)
        pltpu.sync_copy(xs_hbm.at[b, 1], xw_v)

        @plsc.parallel_loop(0, NCHUNKS, unroll=4)
        def idx_body(i):
            sl = pl.ds(i * LANES, LANES)
            dayv = xd_v[sl]
            weekv = xw_v[sl]
            d = jnp.clip((dayv * float(TIME)).astype(jnp.int32), 0, TIME - 1)
            w = jnp.clip(weekv.astype(jnp.int32), 0, 6)
            idxd_v[sl] = d
            idxw_v[sl] = w

        pending = {0: [], 1: []}
        for fblk in range(NBLKS):
            ph = fblk % 2
            for dsc in pending[ph]:
                dsc.wait()
            pending[ph] = []

            @plsc.parallel_loop(0, NCHUNKS, unroll=2)
            def gather_body(i, ph=ph, fblk=fblk):
                sl = pl.ds(i * LANES, LANES)
                dvec = idxd_v[sl]
                wvec = idxw_v[sl]
                nb = i // 8
                off = (i % 8) * LANES
                for j in range(FBLK):
                    f = fblk * FBLK + j
                    dayv = plsc.load_gather(td_v, [dvec + f * TIME])
                    weekv = plsc.load_gather(tw_v, [wvec + f * 7])
                    row_v[ph, j, nb, pl.ds(off, LANES)] = dayv + weekv

            for j in range(FBLK):
                dsc = pltpu.async_copy(
                    row_v.at[ph, j], out_hbm.at[b, fblk, j], sems[ph])
                pending[ph].append(dsc)

        for ph in (0, 1):
            for dsc in pending[ph]:
                dsc.wait()


def kernel(x, time_day, time_week):
    xs = jnp.transpose(x[:, -1, :, 1:3], (0, 2, 1))
    td = time_day.T.reshape(-1)
    tw = time_week.T.reshape(-1)

    mesh = plsc.VectorSubcoreMesh(
        core_axis_name="c", subcore_axis_name="s",
        num_cores=NUM_CORES, num_subcores=NUM_SUBCORES)
    run = functools.partial(
        pl.kernel,
        out_type=jax.ShapeDtypeStruct((B, NBLKS, FBLK, N // 128, 128),
                                      jnp.float32),
        mesh=mesh,
        compiler_params=pltpu.CompilerParams(needs_layout_passes=False),
        scratch_types=[
            pltpu.VMEM((FEATURES * TIME,), jnp.float32),
            pltpu.VMEM((FEATURES * 7,), jnp.float32),
            pltpu.VMEM((N,), jnp.float32),
            pltpu.VMEM((N,), jnp.float32),
            pltpu.VMEM((N,), jnp.int32),
            pltpu.VMEM((N,), jnp.int32),
            pltpu.VMEM((2, FBLK, N // 128, 128), jnp.float32),
            pltpu.SemaphoreType.DMA,
            pltpu.SemaphoreType.DMA,
        ],
    )(_body)
    out = run(xs, td, tw)
    return out.reshape(B, FEATURES, N, 1)


# speedup vs baseline: 3.6802x; 1.0183x over previous
import functools

import jax
import jax.numpy as jnp
from jax import lax
from jax.experimental import pallas as pl
from jax.experimental.pallas import tpu as pltpu
from jax.experimental.pallas import tpu_sc as plsc

TIME = 288
FEATURES = 64
B, T, N, C = 64, 12, 4096, 3

NUM_CORES = 2
NUM_SUBCORES = 16
NUM_WORKERS = NUM_CORES * NUM_SUBCORES
B_PER_W = B // NUM_WORKERS
LANES = 16
NCHUNKS = N // LANES
FBLK = 8
NBLKS = FEATURES // FBLK


def _body(xs_hbm, td_hbm, tw_hbm, out_hbm,
          td_v, tw_v, xd_v, xw_v, idxd_v, idxw_v, row_v, sem0, sem1):
    sems = (sem0, sem1)
    wid = lax.axis_index("s") * NUM_CORES + lax.axis_index("c")

    pltpu.sync_copy(td_hbm, td_v)
    pltpu.sync_copy(tw_hbm, tw_v)

    for b_local in range(B_PER_W):
        b = wid * B_PER_W + b_local

        pltpu.sync_copy(xs_hbm.at[b, 0], xd_v)
        pltpu.sync_copy(xs_hbm.at[b, 1], xw_v)

        @plsc.parallel_loop(0, NCHUNKS, unroll=4)
        def idx_body(i):
            sl = pl.ds(i * LANES, LANES)
            dayv = xd_v[sl]
            weekv = xw_v[sl]
            d = jnp.clip((dayv * float(TIME)).astype(jnp.int32), 0, TIME - 1)
            w = jnp.clip(weekv.astype(jnp.int32), 0, 6)
            idxd_v[sl] = d
            idxw_v[sl] = w

        pending = {0: [], 1: []}
        for fblk in range(NBLKS):
            ph = fblk % 2
            for dsc in pending[ph]:
                dsc.wait()
            pending[ph] = []

            @plsc.parallel_loop(0, NCHUNKS, unroll=4)
            def gather_body(i, ph=ph, fblk=fblk):
                sl = pl.ds(i * LANES, LANES)
                dvec = idxd_v[sl]
                wvec = idxw_v[sl]
                nb = i // 8
                off = (i % 8) * LANES
                for j in range(FBLK):
                    f = fblk * FBLK + j
                    dayv = plsc.load_gather(td_v, [dvec + f * TIME])
                    weekv = plsc.load_gather(tw_v, [wvec + f * 7])
                    row_v[ph, j, nb, pl.ds(off, LANES)] = dayv + weekv

            for j in range(FBLK):
                dsc = pltpu.async_copy(
                    row_v.at[ph, j], out_hbm.at[b, fblk, j], sems[ph])
                pending[ph].append(dsc)

        for ph in (0, 1):
            for dsc in pending[ph]:
                dsc.wait()


def kernel(x, time_day, time_week):
    xs = jnp.transpose(x[:, -1, :, 1:3], (0, 2, 1))
    td = time_day.T.reshape(-1)
    tw = time_week.T.reshape(-1)

    mesh = plsc.VectorSubcoreMesh(
        core_axis_name="c", subcore_axis_name="s",
        num_cores=NUM_CORES, num_subcores=NUM_SUBCORES)
    run = functools.partial(
        pl.kernel,
        out_type=jax.ShapeDtypeStruct((B, NBLKS, FBLK, N // 128, 128),
                                      jnp.float32),
        mesh=mesh,
        compiler_params=pltpu.CompilerParams(needs_layout_passes=False),
        scratch_types=[
            pltpu.VMEM((FEATURES * TIME,), jnp.float32),
            pltpu.VMEM((FEATURES * 7,), jnp.float32),
            pltpu.VMEM((N,), jnp.float32),
            pltpu.VMEM((N,), jnp.float32),
            pltpu.VMEM((N,), jnp.int32),
            pltpu.VMEM((N,), jnp.int32),
            pltpu.VMEM((2, FBLK, N // 128, 128), jnp.float32),
            pltpu.SemaphoreType.DMA,
            pltpu.SemaphoreType.DMA,
        ],
    )(_body)
    out = run(xs, td, tw)
    return out.reshape(B, FEATURES, N, 1)
